# baseline (speedup 1.0000x reference)
import functools

import jax
import jax.numpy as jnp
from jax.experimental import pallas as pl

_E = 8


def _select_top2(l_rows):
    f32 = l_rows[0].dtype
    i32 = jnp.int32
    ninf = jnp.asarray(-jnp.inf, f32)
    g = [jnp.maximum(l_rows[2 * k], l_rows[2 * k + 1]) for k in range(4)]
    m1 = jnp.maximum(jnp.maximum(g[0], g[1]), jnp.maximum(g[2], g[3]))
    gi1 = jnp.where(
        g[0] == m1, 0,
        jnp.where(g[1] == m1, 1, jnp.where(g[2] == m1, 2, 3))).astype(i32)
    ge = [jnp.where(gi1 == k, ninf, g[k]) for k in range(4)]
    m2 = jnp.maximum(jnp.maximum(ge[0], ge[1]), jnp.maximum(ge[2], ge[3]))
    gi2 = jnp.where(
        ge[0] == m2, 0,
        jnp.where(ge[1] == m2, 1, jnp.where(ge[2] == m2, 2, 3))).astype(i32)
    keep = [(gi1 == k) | (gi2 == k) for k in range(4)]
    ms = [jnp.where(keep[e // 2], l_rows[e], ninf) for e in range(8)]
    M1 = ms[0]
    for e in range(1, 8):
        M1 = jnp.maximum(M1, ms[e])
    e1 = jnp.full_like(gi1, 7)
    for e in range(6, -1, -1):
        e1 = jnp.where(ms[e] == M1, e, e1).astype(i32)
    mse = [jnp.where(e1 == e, ninf, ms[e]) for e in range(8)]
    M2 = mse[0]
    for e in range(1, 8):
        M2 = jnp.maximum(M2, mse[e])
    e2 = jnp.full_like(gi1, 7)
    for e in range(6, -1, -1):
        e2 = jnp.where(mse[e] == M2, e, e2).astype(i32)
    return e1, e2, M1, M2


def _gate_block(x_ref, w_ref, idx_ref, wgt_ref):
    lt = jax.lax.dot_general(w_ref[...], x_ref[...], (((1,), (1,)), ((), ())),
                             preferred_element_type=jnp.float32)
    rows = [lt[e:e + 1, :] for e in range(_E)]
    e1, e2, l1, l2 = _select_top2(rows)
    e21 = jnp.exp(l2 - l1)
    q = jnp.asarray(1.0, jnp.float32) / (jnp.asarray(1.0, jnp.float32) + e21)
    idx_ref[...] = jnp.concatenate([e1, e2], axis=0)
    wgt_ref[...] = jnp.concatenate([q, e21 * q], axis=0)


@functools.partial(jax.jit, static_argnames=("block_t",))
def _moe_gate_tc(x, weight, block_t=2048):
    t, h = x.shape
    idx_t, wgt_t = pl.pallas_call(
        _gate_block,
        grid=(t // block_t,),
        in_specs=[
            pl.BlockSpec((block_t, h), lambda i: (i, 0)),
            pl.BlockSpec((weight.shape[0], h), lambda i: (0, 0)),
        ],
        out_specs=[
            pl.BlockSpec((2, block_t), lambda i: (0, i)),
            pl.BlockSpec((2, block_t), lambda i: (0, i)),
        ],
        out_shape=[
            jax.ShapeDtypeStruct((2, t), jnp.int32),
            jax.ShapeDtypeStruct((2, t), jnp.float32),
        ],
    )(x, weight)
    return idx_t.T, wgt_t.T


def kernel(hidden_states, weight):
    bsz, seq_len, h = hidden_states.shape
    x = hidden_states.reshape(-1, h)
    return _moe_gate_tc(x, weight)


# speedup vs baseline: 1.0369x; 1.0369x over previous
import functools

import jax
import jax.numpy as jnp
from jax.experimental import pallas as pl

_E = 8


def _select_top2(l_rows):
    f32 = l_rows[0].dtype
    i32 = jnp.int32
    ninf = jnp.asarray(-jnp.inf, f32)
    g = [jnp.maximum(l_rows[2 * k], l_rows[2 * k + 1]) for k in range(4)]
    m1 = jnp.maximum(jnp.maximum(g[0], g[1]), jnp.maximum(g[2], g[3]))
    gi1 = jnp.where(
        g[0] == m1, 0,
        jnp.where(g[1] == m1, 1, jnp.where(g[2] == m1, 2, 3))).astype(i32)
    ge = [jnp.where(gi1 == k, ninf, g[k]) for k in range(4)]
    m2 = jnp.maximum(jnp.maximum(ge[0], ge[1]), jnp.maximum(ge[2], ge[3]))
    gi2 = jnp.where(
        ge[0] == m2, 0,
        jnp.where(ge[1] == m2, 1, jnp.where(ge[2] == m2, 2, 3))).astype(i32)
    keep = [(gi1 == k) | (gi2 == k) for k in range(4)]
    ms = [jnp.where(keep[e // 2], l_rows[e], ninf) for e in range(8)]
    M1 = ms[0]
    for e in range(1, 8):
        M1 = jnp.maximum(M1, ms[e])
    e1 = jnp.full_like(gi1, 7)
    for e in range(6, -1, -1):
        e1 = jnp.where(ms[e] == M1, e, e1).astype(i32)
    mse = [jnp.where(e1 == e, ninf, ms[e]) for e in range(8)]
    M2 = mse[0]
    for e in range(1, 8):
        M2 = jnp.maximum(M2, mse[e])
    e2 = jnp.full_like(gi1, 7)
    for e in range(6, -1, -1):
        e2 = jnp.where(mse[e] == M2, e, e2).astype(i32)
    return e1, e2, M1, M2


def _gate_block(xa_ref, xb_ref, w_ref, idx_ref, wgt_ref):
    w = w_ref[...]
    lt = jnp.concatenate([
        jax.lax.dot_general(w, xa_ref[...], (((1,), (1,)), ((), ())),
                            preferred_element_type=jnp.float32),
        jax.lax.dot_general(w, xb_ref[...], (((1,), (1,)), ((), ())),
                            preferred_element_type=jnp.float32),
    ], axis=1)
    rows = [lt[e:e + 1, :] for e in range(_E)]
    e1, e2, l1, l2 = _select_top2(rows)
    e21 = jnp.exp(l2 - l1)
    q = jnp.asarray(1.0, jnp.float32) / (jnp.asarray(1.0, jnp.float32) + e21)
    idx_ref[...] = jnp.concatenate([e1, e2], axis=0)
    wgt_ref[...] = jnp.concatenate([q, e21 * q], axis=0)


@functools.partial(jax.jit, static_argnames=("block_t",))
def _moe_gate_tc(x, weight, block_t=1024):
    t, h = x.shape
    idx_t, wgt_t = pl.pallas_call(
        _gate_block,
        grid=(t // block_t,),
        in_specs=[
            pl.BlockSpec((block_t // 2, h), lambda i: (2 * i, 0)),
            pl.BlockSpec((block_t // 2, h), lambda i: (2 * i + 1, 0)),
            pl.BlockSpec((weight.shape[0], h), lambda i: (0, 0)),
        ],
        out_specs=[
            pl.BlockSpec((2, block_t), lambda i: (0, i)),
            pl.BlockSpec((2, block_t), lambda i: (0, i)),
        ],
        out_shape=[
            jax.ShapeDtypeStruct((2, t), jnp.int32),
            jax.ShapeDtypeStruct((2, t), jnp.float32),
        ],
    )(x, x, weight)
    return idx_t.T, wgt_t.T


def kernel(hidden_states, weight):
    bsz, seq_len, h = hidden_states.shape
    x = hidden_states.reshape(-1, h)
    return _moe_gate_tc(x, weight)
